# Initial kernel scaffold; baseline (speedup 1.0000x reference)
import jax
import jax.numpy as jnp
from jax.experimental import pallas as pl


def kernel(X, edge_index, hg_v_idx, hg_e_idx, num_hyperedges, W_theta, b_theta, a_src, a_dst):
    raise NotImplementedError("write your pallas kernel here")


# speedup vs baseline: 1.2716x; 1.2716x over previous
import functools

import jax
import jax.numpy as jnp
from jax.experimental import pallas as pl
from jax.experimental.pallas import tpu as pltpu


def _dense_body(x_ref, w_ref, b_ref, asrc_ref, adst_ref, xo_ref, s_ref, c_ref):
    xo = jnp.dot(x_ref[...], w_ref[...].T, preferred_element_type=jnp.float32)
    xo = xo + b_ref[...]
    xo_ref[...] = xo
    s_src = jnp.dot(xo, asrc_ref[...].T, preferred_element_type=jnp.float32)
    s_dst = jnp.dot(xo, adst_ref[...].T, preferred_element_type=jnp.float32)
    s_ref[0, :, :] = s_src.T
    s_ref[1, :, :] = s_dst.T
    c_ref[...] = jnp.full((1, 128), jnp.max(s_src) + jnp.max(s_dst), jnp.float32)


def _dense_stage(X, W_theta, b_theta, a_src, a_dst):
    N, d = X.shape
    out_shapes = (
        jax.ShapeDtypeStruct((N, d), jnp.float32),
        jax.ShapeDtypeStruct((2, 1, N), jnp.float32),
        jax.ShapeDtypeStruct((1, 128), jnp.float32),
    )
    return pl.pallas_call(
        _dense_body,
        out_shape=out_shapes,
    )(X, W_theta, b_theta.reshape(1, d), a_src.reshape(1, d), a_dst.reshape(1, d))


def kernel(X, edge_index, hg_v_idx, hg_e_idx, num_hyperedges, W_theta, b_theta, a_src, a_dst):
    N = X.shape[0]
    NHE = 5000
    scale = jnp.asarray(num_hyperedges, jnp.float32) / NHE
    neg_slope = 0.2

    Xo, s, c = _dense_stage(X, W_theta, b_theta, a_src, a_dst)
    s_src = s[0, 0]
    s_dst = s[1, 0]
    c = c[0, 0]

    e_src = edge_index[0]
    e_dst = edge_index[1]
    score = s_src[e_src] + s_dst[e_dst]
    score = jnp.where(score >= 0, score, neg_slope * score)
    ex = jnp.exp(score - jnp.maximum(c, neg_slope * c))
    denom = jax.ops.segment_sum(ex, e_dst, num_segments=N)
    alpha = ex / denom[e_dst]
    X_g = jax.ops.segment_sum(alpha[:, None] * Xo[e_src], e_dst, num_segments=N)

    dv = jax.ops.segment_sum(jnp.ones(hg_v_idx.shape[0], dtype=jnp.float32), hg_v_idx, num_segments=N)
    de = jax.ops.segment_sum(jnp.ones(hg_e_idx.shape[0], dtype=jnp.float32), hg_e_idx, num_segments=NHE)
    dv_isqrt = jnp.where(dv > 0, 1.0 / jnp.sqrt(jnp.maximum(dv, 1e-12)), 0.0)
    de_inv = jnp.where(de > 0, scale / jnp.maximum(de, 1e-12), 0.0)
    Y = Xo * dv_isqrt[:, None]
    M = jax.ops.segment_sum(Y[hg_v_idx], hg_e_idx, num_segments=NHE)
    M = M * de_inv[:, None]
    Z = jax.ops.segment_sum(M[hg_e_idx], hg_v_idx, num_segments=N)
    X_hg = Z * dv_isqrt[:, None]
    X1 = (X_g + X_hg) / 2.0
    return jax.nn.elu(X1)


# speedup vs baseline: 18.3120x; 14.4008x over previous
import functools

import jax
import jax.numpy as jnp
from jax import lax
from jax.experimental import pallas as pl
from jax.experimental.pallas import tpu as pltpu
from jax.experimental.pallas import tpu_sc as plsc

NC = 2
NS = 16
NW = NC * NS
L = 16
NEG_SLOPE = 0.2


def _ceil_to(x, m):
    return -(-x // m) * m


def _dense_body(x_ref, w_ref, b_ref, asrc_ref, adst_ref, xo_ref, s_ref, c_ref):
    xo = jnp.dot(x_ref[...], w_ref[...].T, preferred_element_type=jnp.float32)
    xo = xo + b_ref[...]
    xo_ref[...] = xo
    s_src = jnp.dot(xo, asrc_ref[...].T, preferred_element_type=jnp.float32)
    s_dst = jnp.dot(xo, adst_ref[...].T, preferred_element_type=jnp.float32)
    s_ref[0, :, :] = s_src.T
    s_ref[1, :, :] = s_dst.T
    c_ref[...] = jnp.full((1, 128), jnp.max(s_src) + jnp.max(s_dst), jnp.float32)


def _dense_stage(X, W_theta, b_theta, a_src, a_dst):
    N, d = X.shape
    out_shapes = (
        jax.ShapeDtypeStruct((N, d), jnp.float32),
        jax.ShapeDtypeStruct((2, 1, N), jnp.float32),
        jax.ShapeDtypeStruct((1, 128), jnp.float32),
    )
    return pl.pallas_call(_dense_body, out_shape=out_shapes)(
        X, W_theta, b_theta.reshape(1, d), a_src.reshape(1, d), a_dst.reshape(1, d))


@functools.lru_cache(maxsize=None)
def _build_edge_scalar(N, E, NNZ, NHE):
    EPT = E // NW
    GE = EPT // L
    CPT = NNZ // NW
    WLEN = _ceil_to(CPT + 7, 2 * L)
    GH = WLEN // L

    mesh = plsc.VectorSubcoreMesh(core_axis_name="c", subcore_axis_name="s")
    out_type = (
        jax.ShapeDtypeStruct((E,), jnp.float32),
        jax.ShapeDtypeStruct((NW, 1, N), jnp.float32),
        jax.ShapeDtypeStruct((NW, 1, N), jnp.float32),
        jax.ShapeDtypeStruct((NW, 1, NHE), jnp.float32),
    )
    scratch = [
        pltpu.VMEM((N,), jnp.float32),
        pltpu.VMEM((N,), jnp.float32),
        pltpu.VMEM((EPT,), jnp.int32),
        pltpu.VMEM((EPT,), jnp.int32),
        pltpu.VMEM((EPT,), jnp.float32),
        pltpu.VMEM((N,), jnp.float32),
        pltpu.VMEM((WLEN,), jnp.int32),
        pltpu.VMEM((WLEN,), jnp.int32),
        pltpu.VMEM((NHE,), jnp.float32),
        pltpu.VMEM((L,), jnp.float32),
    ]

    @functools.partial(pl.kernel, out_type=out_type, mesh=mesh,
                       scratch_types=scratch,
                       compiler_params=pltpu.CompilerParams(
                           needs_layout_passes=False))
    def k(ssrc_h, sdst_h, c_h, esrc_h, edst_h, hgv_h, hge_h,
          ex_h, den_h, dv_h, de_h,
          ssrc_v, sdst_v, esrc_v, edst_v, ex_v, acc_v, hgv_v, hge_v, de_v, c_v):
        cid = lax.axis_index("c")
        sid = lax.axis_index("s")
        wid = sid * NC + cid
        base = pl.multiple_of(wid * EPT, 8)

        pltpu.sync_copy(ssrc_h, ssrc_v)
        pltpu.sync_copy(sdst_h, sdst_v)
        pltpu.sync_copy(c_h, c_v)
        pltpu.sync_copy(esrc_h.at[pl.ds(base, EPT)], esrc_v)
        pltpu.sync_copy(edst_h.at[pl.ds(base, EPT)], edst_v)

        zf = jnp.zeros((L,), jnp.float32)

        def zero_acc(i, _):
            acc_v[pl.ds(i * L, L)] = zf
            return _
        lax.fori_loop(0, N // L, zero_acc, None)

        cvec = jnp.full((L,), c_v[pl.ds(0, L)][0], jnp.float32)

        def ebody(g, _):
            off = pl.multiple_of(g * L, 8)
            si = esrc_v[pl.ds(off, L)]
            di = edst_v[pl.ds(off, L)]
            vs = plsc.load_gather(ssrc_v, [si])
            vd = plsc.load_gather(sdst_v, [di])
            sc = vs + vd
            sc = jnp.where(sc >= 0, sc, jnp.float32(NEG_SLOPE) * sc)
            ex = jnp.exp(sc - cvec)
            ex_v[pl.ds(off, L)] = ex
            plsc.addupdate_scatter(acc_v, [di], ex)
            return _
        lax.fori_loop(0, GE, ebody, None)

        pltpu.sync_copy(ex_v, ex_h.at[pl.ds(base, EPT)])
        pltpu.sync_copy(acc_v, den_h.at[wid, 0])

        lo = wid * CPT
        hi = lo + CPT
        st = pl.multiple_of(jnp.minimum((lo // 8) * 8, NNZ - WLEN), 8)
        pltpu.sync_copy(hgv_h.at[pl.ds(st, WLEN)], hgv_v)
        pltpu.sync_copy(hge_h.at[pl.ds(st, WLEN)], hge_v)

        lax.fori_loop(0, N // L, zero_acc, None)

        def zero_de(i, _):
            de_v[pl.ds(i * L, L)] = zf
            return _
        lax.fori_loop(0, NHE // L, zero_de, None)

        iota = lax.iota(jnp.int32, L)
        ones = jnp.ones((L,), jnp.float32)

        def hbody(g, _):
            off = pl.multiple_of(g * L, 8)
            pos = st + off + iota
            m = (pos >= lo) & (pos < hi)
            vi = hgv_v[pl.ds(off, L)]
            ve = hge_v[pl.ds(off, L)]
            plsc.addupdate_scatter(acc_v, [vi], ones, mask=m)
            plsc.addupdate_scatter(de_v, [ve], ones, mask=m)
            return _
        lax.fori_loop(0, GH, hbody, None)

        pltpu.sync_copy(acc_v, dv_h.at[wid, 0])
        pltpu.sync_copy(de_v, de_h.at[wid, 0])

    return k


def _r1_body(denp_ref, dvp_ref, dep_ref, xo_ref,
             rden_ref, dvi_ref, dei_ref, y_ref):
    den = jnp.sum(denp_ref[...], axis=(0, 1))
    rden_ref[...] = jnp.where(den > 0, 1.0 / den, 0.0)[None, :]
    dv = jnp.sum(dvp_ref[...], axis=(0, 1))
    dvi = jnp.where(dv > 0, lax.rsqrt(jnp.maximum(dv, 1e-12)), 0.0)
    dvi_ref[...] = dvi[None, :]
    de = jnp.sum(dep_ref[...], axis=(0, 1))
    dei_ref[...] = jnp.where(de > 0, 1.0 / jnp.maximum(de, 1e-12), 0.0)[None, :]
    y_ref[...] = xo_ref[...] * dvi[:, None]


def _r1_stage(den_p, dv_p, de_p, Xo):
    N, d = Xo.shape
    NHE = de_p.shape[-1]
    out_shapes = (
        jax.ShapeDtypeStruct((1, N), jnp.float32),
        jax.ShapeDtypeStruct((1, N), jnp.float32),
        jax.ShapeDtypeStruct((1, NHE), jnp.float32),
        jax.ShapeDtypeStruct((N, d), jnp.float32),
    )
    return pl.pallas_call(_r1_body, out_shape=out_shapes)(den_p, dv_p, de_p, Xo)


@functools.lru_cache(maxsize=None)
def _build_edge_agg(N, E, D):
    EPT = E // NW
    GK = 80
    CE = 2000
    NCH = EPT // CE
    NG = CE // GK
    NPAD = _ceil_to(N, NS * 8)
    RPS = NPAD // NS
    ZR = 16
    ZFULL, ZREM = RPS // ZR, RPS % ZR

    mesh = plsc.VectorSubcoreMesh(core_axis_name="c", subcore_axis_name="s")
    out_type = jax.ShapeDtypeStruct((NC, NPAD, D), jnp.float32)
    scratch = [
        pltpu.VMEM_SHARED((NPAD, D), jnp.float32),
        pltpu.VMEM((CE,), jnp.int32),
        pltpu.VMEM((CE,), jnp.int32),
        pltpu.VMEM((CE,), jnp.float32),
        pltpu.VMEM((N,), jnp.float32),
        pltpu.VMEM((GK, D), jnp.float32),
        pltpu.VMEM((GK,), jnp.int32),
        pltpu.VMEM((GK,), jnp.float32),
        pltpu.VMEM((ZR, D), jnp.float32),
        pltpu.SemaphoreType.DMA,
    ]

    @functools.partial(pl.kernel, out_type=out_type, mesh=mesh,
                       scratch_types=scratch,
                       compiler_params=pltpu.CompilerParams(
                           needs_layout_passes=False))
    def k(xo_h, ex_h, rden_h, esrc_h, edst_h,
          xg_h,
          acc_sh, esrc_v, edst_v, ex_v, rden_v, rows_v, dst_v, alph_v, zb_v, sem):
        cid = lax.axis_index("c")
        sid = lax.axis_index("s")
        wid = sid * NC + cid
        base = pl.multiple_of(wid * EPT, 8)

        pltpu.sync_copy(rden_h.at[0], rden_v)

        zf = jnp.zeros((L,), jnp.float32)

        def zero_zb(r, _):
            for k2 in range(D // L):
                zb_v[r, pl.ds(k2 * L, L)] = zf
            return _
        lax.fori_loop(0, ZR, zero_zb, None)

        def zero_acc(t, _):
            pltpu.sync_copy(zb_v, acc_sh.at[pl.ds(sid * RPS + t * ZR, ZR)])
            return _
        lax.fori_loop(0, ZFULL, zero_acc, None)
        if ZREM:
            pltpu.sync_copy(zb_v.at[pl.ds(0, ZREM)],
                            acc_sh.at[pl.ds(sid * RPS + ZFULL * ZR, ZREM)])
        plsc.subcore_barrier()

        def cbody(ch, _):
            cb = pl.multiple_of(base + ch * CE, 8)
            pltpu.sync_copy(esrc_h.at[pl.ds(cb, CE)], esrc_v)
            pltpu.sync_copy(edst_h.at[pl.ds(cb, CE)], edst_v)
            pltpu.sync_copy(ex_h.at[pl.ds(cb, CE)], ex_v)

            def gbody(g, _g):
                gb = pl.multiple_of(g * GK, 8)
                cp = pltpu.async_copy(xo_h.at[esrc_v.at[pl.ds(gb, GK)]],
                                      rows_v, sem)
                for sub in range(GK // L):
                    off = pl.multiple_of(gb + sub * L, 8)
                    di = edst_v[pl.ds(off, L)]
                    exv = ex_v[pl.ds(off, L)]
                    rd = plsc.load_gather(rden_v, [di])
                    alph_v[pl.ds(sub * L, L)] = exv * rd
                    dst_v[pl.ds(sub * L, L)] = di
                cp.wait()
                for sub in range(GK // L):
                    avec = alph_v[pl.ds(sub * L, L)]
                    for jj in range(L):
                        j = sub * L + jj
                        av = jnp.full((L,), avec[jj], jnp.float32)
                        for k2 in range(D // L):
                            rows_v[j, pl.ds(k2 * L, L)] = (
                                rows_v[j, pl.ds(k2 * L, L)] * av)
                pltpu.sync_copy(rows_v, acc_sh.at[dst_v], add=True)
                return _g
            lax.fori_loop(0, NG, gbody, None)
            return _
        lax.fori_loop(0, NCH, cbody, None)

        plsc.subcore_barrier()
        pltpu.sync_copy(acc_sh.at[pl.ds(sid * RPS, RPS)],
                        xg_h.at[cid, pl.ds(sid * RPS, RPS)])

    return k


@functools.lru_cache(maxsize=None)
def _build_gather_scatter(T, A, APAD, NNZ, D):
    CPT = NNZ // NW
    GK = 64
    WLEN = _ceil_to(CPT + 7, GK)
    NGH = WLEN // GK
    RPS = APAD // NS
    ZR = 16
    ZFULL, ZREM = RPS // ZR, RPS % ZR

    mesh = plsc.VectorSubcoreMesh(core_axis_name="c", subcore_axis_name="s")
    out_type = jax.ShapeDtypeStruct((NC, APAD, D), jnp.float32)
    scratch = [
        pltpu.VMEM_SHARED((APAD, D), jnp.float32),
        pltpu.VMEM((WLEN,), jnp.int32),
        pltpu.VMEM((WLEN,), jnp.int32),
        pltpu.VMEM((GK, D), jnp.float32),
        pltpu.VMEM((GK,), jnp.int32),
        pltpu.VMEM((ZR, D), jnp.float32),
        pltpu.SemaphoreType.DMA,
    ]

    @functools.partial(pl.kernel, out_type=out_type, mesh=mesh,
                       scratch_types=scratch,
                       compiler_params=pltpu.CompilerParams(
                           needs_layout_passes=False))
    def k(tab_h, src_h, dst_h,
          out_h,
          acc_sh, src_v, dst_v, rows_v, db_v, zb_v, sem):
        cid = lax.axis_index("c")
        sid = lax.axis_index("s")
        wid = sid * NC + cid
        lo = wid * CPT
        hi = lo + CPT
        st = pl.multiple_of(jnp.minimum((lo // 8) * 8, NNZ - WLEN), 8)

        pltpu.sync_copy(src_h.at[pl.ds(st, WLEN)], src_v)
        pltpu.sync_copy(dst_h.at[pl.ds(st, WLEN)], dst_v)

        zf = jnp.zeros((L,), jnp.float32)

        def zero_zb(r, _):
            for k2 in range(D // L):
                zb_v[r, pl.ds(k2 * L, L)] = zf
            return _
        lax.fori_loop(0, ZR, zero_zb, None)

        def zero_acc(t, _):
            pltpu.sync_copy(zb_v, acc_sh.at[pl.ds(sid * RPS + t * ZR, ZR)])
            return _
        lax.fori_loop(0, ZFULL, zero_acc, None)
        if ZREM:
            pltpu.sync_copy(zb_v.at[pl.ds(0, ZREM)],
                            acc_sh.at[pl.ds(sid * RPS + ZFULL * ZR, ZREM)])
        plsc.subcore_barrier()

        iota = lax.iota(jnp.int32, L)

        def gbody(g, _):
            gb = pl.multiple_of(g * GK, 8)
            cp = pltpu.async_copy(tab_h.at[src_v.at[pl.ds(gb, GK)]], rows_v, sem)
            for sub in range(GK // L):
                off = pl.multiple_of(gb + sub * L, 8)
                pos = st + off + iota
                m = (pos >= lo) & (pos < hi)
                dd = dst_v[pl.ds(off, L)]
                db_v[pl.ds(sub * L, L)] = jnp.where(m, dd, jnp.int32(A))
            cp.wait()
            pltpu.sync_copy(rows_v, acc_sh.at[db_v], add=True)
            return _
        lax.fori_loop(0, NGH, gbody, None)

        plsc.subcore_barrier()
        pltpu.sync_copy(acc_sh.at[pl.ds(sid * RPS, RPS)],
                        out_h.at[cid, pl.ds(sid * RPS, RPS)])

    return k


def _r2_body(mp_ref, dei_ref, m_ref):
    nhe = m_ref.shape[0]
    m = mp_ref[0, :nhe, :] + mp_ref[1, :nhe, :]
    m_ref[...] = m * dei_ref[0, :][:, None]


def _r2_stage(m_p, dei, NHE, D):
    return pl.pallas_call(
        _r2_body,
        out_shape=jax.ShapeDtypeStruct((NHE, D), jnp.float32),
    )(m_p, dei)


def _f_body(xgp_ref, zgp_ref, dvi_ref, scale_ref, out_ref):
    n = out_ref.shape[0]
    xg = xgp_ref[0, :n, :] + xgp_ref[1, :n, :]
    z = zgp_ref[0, :n, :] + zgp_ref[1, :n, :]
    xhg = z * dvi_ref[0, :][:, None] * scale_ref[...]
    x1 = (xg + xhg) * 0.5
    out_ref[...] = jnp.where(x1 > 0, x1, jnp.exp(jnp.minimum(x1, 0.0)) - 1.0)


def _f_stage(xg_p, zg_p, dvi, scale_row, N, D):
    return pl.pallas_call(
        _f_body,
        out_shape=jax.ShapeDtypeStruct((N, D), jnp.float32),
    )(xg_p, zg_p, dvi, scale_row)


def kernel(X, edge_index, hg_v_idx, hg_e_idx, num_hyperedges, W_theta, b_theta,
           a_src, a_dst):
    N, D = X.shape
    E = edge_index.shape[1]
    NNZ = hg_v_idx.shape[0]
    NHE = 5000
    NPAD = _ceil_to(N + 1, NS * 8)
    HPAD = _ceil_to(NHE + 1, NS * 8)

    scale_row = jnp.broadcast_to(
        (jnp.asarray(num_hyperedges, jnp.float32) / NHE)[None, None], (1, D))

    Xo, s, c = _dense_stage(X, W_theta, b_theta, a_src, a_dst)
    e_src = edge_index[0]
    e_dst = edge_index[1]

    k2 = _build_edge_scalar(N, E, NNZ, NHE)
    ex, den_p, dv_p, de_p = k2(s[0, 0], s[1, 0], c[0, :L], e_src, e_dst,
                               hg_v_idx, hg_e_idx)

    rden, dvi, dei, Y = _r1_stage(den_p, dv_p, de_p, Xo)

    k3 = _build_edge_agg(N, E, D)
    xg_p = k3(Xo, ex, rden, e_src, e_dst)

    k4 = _build_gather_scatter(N, NHE, HPAD, NNZ, D)
    m_p = k4(Y, hg_v_idx, hg_e_idx)

    M = _r2_stage(m_p, dei, NHE, D)

    k5 = _build_gather_scatter(NHE, N, NPAD, NNZ, D)
    zg_p = k5(M, hg_e_idx, hg_v_idx)

    return _f_stage(xg_p, zg_p, dvi, scale_row, N, D)


# speedup vs baseline: 26.2060x; 1.4311x over previous
import functools

import jax
import jax.numpy as jnp
from jax import lax
from jax.experimental import pallas as pl
from jax.experimental.pallas import tpu as pltpu
from jax.experimental.pallas import tpu_sc as plsc

NC = 2
NS = 16
NW = NC * NS
L = 16
NEG_SLOPE = 0.2


def _ceil_to(x, m):
    return -(-x // m) * m


def _dense_body(x_ref, w_ref, b_ref, asrc_ref, adst_ref, xo_ref, s_ref, c_ref):
    xo = jnp.dot(x_ref[...], w_ref[...].T, preferred_element_type=jnp.float32)
    xo = xo + b_ref[...]
    xo_ref[...] = xo
    s_src = jnp.dot(xo, asrc_ref[...].T, preferred_element_type=jnp.float32)
    s_dst = jnp.dot(xo, adst_ref[...].T, preferred_element_type=jnp.float32)
    s_ref[0, :, :] = s_src.T
    s_ref[1, :, :] = s_dst.T
    c_ref[...] = jnp.full((1, 128), jnp.max(s_src) + jnp.max(s_dst), jnp.float32)


def _dense_stage(X, W_theta, b_theta, a_src, a_dst):
    N, d = X.shape
    out_shapes = (
        jax.ShapeDtypeStruct((N, d), jnp.float32),
        jax.ShapeDtypeStruct((2, 1, N), jnp.float32),
        jax.ShapeDtypeStruct((1, 128), jnp.float32),
    )
    return pl.pallas_call(_dense_body, out_shape=out_shapes)(
        X, W_theta, b_theta.reshape(1, d), a_src.reshape(1, d), a_dst.reshape(1, d))


@functools.lru_cache(maxsize=None)
def _build_edge_scalar(N, E, NNZ, NHE):
    EPT = E // NW
    GE = EPT // L
    CPT = NNZ // NW
    WLEN = _ceil_to(CPT + 7, 2 * L)
    GH = WLEN // L

    mesh = plsc.VectorSubcoreMesh(core_axis_name="c", subcore_axis_name="s")
    out_type = (
        jax.ShapeDtypeStruct((E,), jnp.float32),
        jax.ShapeDtypeStruct((NW, 1, N), jnp.float32),
        jax.ShapeDtypeStruct((NW, 1, N), jnp.float32),
        jax.ShapeDtypeStruct((NW, 1, NHE), jnp.float32),
    )
    scratch = [
        pltpu.VMEM((N,), jnp.float32),
        pltpu.VMEM((N,), jnp.float32),
        pltpu.VMEM((EPT,), jnp.int32),
        pltpu.VMEM((EPT,), jnp.int32),
        pltpu.VMEM((EPT,), jnp.float32),
        pltpu.VMEM((N,), jnp.float32),
        pltpu.VMEM((WLEN,), jnp.int32),
        pltpu.VMEM((WLEN,), jnp.int32),
        pltpu.VMEM((NHE,), jnp.float32),
        pltpu.VMEM((L,), jnp.float32),
    ]

    @functools.partial(pl.kernel, out_type=out_type, mesh=mesh,
                       scratch_types=scratch,
                       compiler_params=pltpu.CompilerParams(
                           needs_layout_passes=False))
    def k(ssrc_h, sdst_h, c_h, esrc_h, edst_h, hgv_h, hge_h,
          ex_h, den_h, dv_h, de_h,
          ssrc_v, sdst_v, esrc_v, edst_v, ex_v, acc_v, hgv_v, hge_v, de_v, c_v):
        cid = lax.axis_index("c")
        sid = lax.axis_index("s")
        wid = sid * NC + cid
        base = pl.multiple_of(wid * EPT, 8)

        pltpu.sync_copy(ssrc_h, ssrc_v)
        pltpu.sync_copy(sdst_h, sdst_v)
        pltpu.sync_copy(c_h, c_v)
        pltpu.sync_copy(esrc_h.at[pl.ds(base, EPT)], esrc_v)
        pltpu.sync_copy(edst_h.at[pl.ds(base, EPT)], edst_v)

        zf = jnp.zeros((L,), jnp.float32)

        def zero_acc(i, _):
            acc_v[pl.ds(i * L, L)] = zf
            return _
        lax.fori_loop(0, N // L, zero_acc, None)

        cvec = jnp.full((L,), c_v[pl.ds(0, L)][0], jnp.float32)

        def ebody(g, _):
            off = pl.multiple_of(g * L, 8)
            si = esrc_v[pl.ds(off, L)]
            di = edst_v[pl.ds(off, L)]
            vs = plsc.load_gather(ssrc_v, [si])
            vd = plsc.load_gather(sdst_v, [di])
            sc = vs + vd
            sc = jnp.where(sc >= 0, sc, jnp.float32(NEG_SLOPE) * sc)
            ex = jnp.exp(sc - cvec)
            ex_v[pl.ds(off, L)] = ex
            plsc.addupdate_scatter(acc_v, [di], ex)
            return _
        lax.fori_loop(0, GE, ebody, None)

        pltpu.sync_copy(ex_v, ex_h.at[pl.ds(base, EPT)])
        pltpu.sync_copy(acc_v, den_h.at[wid, 0])

        lo = wid * CPT
        hi = lo + CPT
        st = pl.multiple_of(jnp.minimum((lo // 8) * 8, NNZ - WLEN), 8)
        pltpu.sync_copy(hgv_h.at[pl.ds(st, WLEN)], hgv_v)
        pltpu.sync_copy(hge_h.at[pl.ds(st, WLEN)], hge_v)

        lax.fori_loop(0, N // L, zero_acc, None)

        def zero_de(i, _):
            de_v[pl.ds(i * L, L)] = zf
            return _
        lax.fori_loop(0, NHE // L, zero_de, None)

        iota = lax.iota(jnp.int32, L)
        ones = jnp.ones((L,), jnp.float32)

        def hbody(g, _):
            off = pl.multiple_of(g * L, 8)
            pos = st + off + iota
            m = (pos >= lo) & (pos < hi)
            vi = hgv_v[pl.ds(off, L)]
            ve = hge_v[pl.ds(off, L)]
            plsc.addupdate_scatter(acc_v, [vi], ones, mask=m)
            plsc.addupdate_scatter(de_v, [ve], ones, mask=m)
            return _
        lax.fori_loop(0, GH, hbody, None)

        pltpu.sync_copy(acc_v, dv_h.at[wid, 0])
        pltpu.sync_copy(de_v, de_h.at[wid, 0])

    return k


def _r1_body(denp_ref, dvp_ref, dep_ref, xo_ref,
             rden_ref, dvi_ref, dei_ref, y_ref):
    den = jnp.sum(denp_ref[...], axis=(0, 1))
    rden_ref[...] = jnp.where(den > 0, 1.0 / den, 0.0)[None, :]
    dv = jnp.sum(dvp_ref[...], axis=(0, 1))
    dvi = jnp.where(dv > 0, lax.rsqrt(jnp.maximum(dv, 1e-12)), 0.0)
    dvi_ref[...] = dvi[None, :]
    de = jnp.sum(dep_ref[...], axis=(0, 1))
    dei_ref[...] = jnp.where(de > 0, 1.0 / jnp.maximum(de, 1e-12), 0.0)[None, :]
    y_ref[...] = xo_ref[...] * dvi[:, None]


def _r1_stage(den_p, dv_p, de_p, Xo):
    N, d = Xo.shape
    NHE = de_p.shape[-1]
    out_shapes = (
        jax.ShapeDtypeStruct((1, N), jnp.float32),
        jax.ShapeDtypeStruct((1, N), jnp.float32),
        jax.ShapeDtypeStruct((1, NHE), jnp.float32),
        jax.ShapeDtypeStruct((N, d), jnp.float32),
    )
    return pl.pallas_call(_r1_body, out_shape=out_shapes)(den_p, dv_p, de_p, Xo)


@functools.lru_cache(maxsize=None)
def _build_edge_agg(N, E, D):
    EPT = E // NW
    GK = 80
    CE = 2000
    NCH = EPT // CE
    NG = CE // GK
    NPAD = _ceil_to(N, NS * 8)
    RPS = NPAD // NS
    ZR = 16
    ZFULL, ZREM = RPS // ZR, RPS % ZR

    NP = NG // 2
    assert NG == 2 * NP + 1

    mesh = plsc.VectorSubcoreMesh(core_axis_name="c", subcore_axis_name="s")
    out_type = jax.ShapeDtypeStruct((NC, NPAD, D), jnp.float32)
    scratch = [
        pltpu.VMEM_SHARED((NPAD, D), jnp.float32),
        pltpu.VMEM((CE,), jnp.int32),
        pltpu.VMEM((CE,), jnp.int32),
        pltpu.VMEM((CE,), jnp.float32),
        pltpu.VMEM((N,), jnp.float32),
        pltpu.VMEM((GK, D), jnp.float32),
        pltpu.VMEM((GK, D), jnp.float32),
        pltpu.VMEM((GK,), jnp.int32),
        pltpu.VMEM((GK,), jnp.int32),
        pltpu.VMEM((GK,), jnp.float32),
        pltpu.VMEM((GK,), jnp.float32),
        pltpu.VMEM((ZR, D), jnp.float32),
        pltpu.SemaphoreType.DMA,
        pltpu.SemaphoreType.DMA,
    ]

    @functools.partial(pl.kernel, out_type=out_type, mesh=mesh,
                       scratch_types=scratch,
                       compiler_params=pltpu.CompilerParams(
                           needs_layout_passes=False))
    def k(xo_h, ex_h, rden_h, esrc_h, edst_h,
          xg_h,
          acc_sh, esrc_v, edst_v, ex_v, rden_v, rows0_v, rows1_v,
          dst0_v, dst1_v, alph0_v, alph1_v, zb_v, sem0, sem1):
        cid = lax.axis_index("c")
        sid = lax.axis_index("s")
        wid = sid * NC + cid
        base = pl.multiple_of(wid * EPT, 8)

        pltpu.sync_copy(rden_h.at[0], rden_v)

        zf = jnp.zeros((L,), jnp.float32)

        def zero_zb(r, _):
            for k2 in range(D // L):
                zb_v[r, pl.ds(k2 * L, L)] = zf
            return _
        lax.fori_loop(0, ZR, zero_zb, None)

        def zero_acc(t, _):
            pltpu.sync_copy(zb_v, acc_sh.at[pl.ds(sid * RPS + t * ZR, ZR)])
            return _
        lax.fori_loop(0, ZFULL, zero_acc, None)
        if ZREM:
            pltpu.sync_copy(zb_v.at[pl.ds(0, ZREM)],
                            acc_sh.at[pl.ds(sid * RPS + ZFULL * ZR, ZREM)])
        plsc.subcore_barrier()

        def prep(g, alph_b, dst_b):
            gb = pl.multiple_of(g * GK, 8)
            for sub in range(GK // L):
                off = pl.multiple_of(gb + sub * L, 8)
                di = edst_v[pl.ds(off, L)]
                exv = ex_v[pl.ds(off, L)]
                rd = plsc.load_gather(rden_v, [di])
                alph_b[pl.ds(sub * L, L)] = exv * rd
                dst_b[pl.ds(sub * L, L)] = di

        def issue(g, rows_b, sem_b):
            gb = pl.multiple_of(g * GK, 8)
            return pltpu.async_copy(xo_h.at[esrc_v.at[pl.ds(gb, GK)]],
                                    rows_b, sem_b)

        def wait0():
            pltpu.make_async_copy(xo_h.at[pl.ds(0, GK)], rows0_v, sem0).wait()

        def scale_scat(rows_b, alph_b, dst_b):
            for sub in range(GK // L):
                avec = alph_b[pl.ds(sub * L, L)]
                for jj in range(L):
                    j = sub * L + jj
                    av = jnp.full((L,), avec[jj], jnp.float32)
                    for k2 in range(D // L):
                        rows_b[j, pl.ds(k2 * L, L)] = (
                            rows_b[j, pl.ds(k2 * L, L)] * av)
            pltpu.sync_copy(rows_b, acc_sh.at[dst_b], add=True)

        def cbody(ch, _):
            cb = pl.multiple_of(base + ch * CE, 8)
            pltpu.sync_copy(esrc_h.at[pl.ds(cb, CE)], esrc_v)
            pltpu.sync_copy(edst_h.at[pl.ds(cb, CE)], edst_v)
            pltpu.sync_copy(ex_h.at[pl.ds(cb, CE)], ex_v)

            prep(0, alph0_v, dst0_v)
            issue(0, rows0_v, sem0)

            def pbody(u, _u):
                g1 = 2 * u + 1
                prep(g1, alph1_v, dst1_v)
                cp1 = issue(g1, rows1_v, sem1)
                wait0()
                scale_scat(rows0_v, alph0_v, dst0_v)
                prep(g1 + 1, alph0_v, dst0_v)
                issue(g1 + 1, rows0_v, sem0)
                cp1.wait()
                scale_scat(rows1_v, alph1_v, dst1_v)
                return _u
            lax.fori_loop(0, NP, pbody, None)

            wait0()
            scale_scat(rows0_v, alph0_v, dst0_v)
            return _
        lax.fori_loop(0, NCH, cbody, None)

        plsc.subcore_barrier()
        pltpu.sync_copy(acc_sh.at[pl.ds(sid * RPS, RPS)],
                        xg_h.at[cid, pl.ds(sid * RPS, RPS)])

    return k


@functools.lru_cache(maxsize=None)
def _build_gather_scatter(T, A, APAD, NNZ, D):
    CPT = NNZ // NW
    GK = 64
    WLEN = _ceil_to(CPT + 7, 2 * GK)
    NGH = WLEN // GK
    NP = NGH // 2
    assert NGH == 2 * NP
    RPS = APAD // NS
    ZR = 16
    ZFULL, ZREM = RPS // ZR, RPS % ZR

    mesh = plsc.VectorSubcoreMesh(core_axis_name="c", subcore_axis_name="s")
    out_type = jax.ShapeDtypeStruct((NC, APAD, D), jnp.float32)
    scratch = [
        pltpu.VMEM_SHARED((APAD, D), jnp.float32),
        pltpu.VMEM((WLEN,), jnp.int32),
        pltpu.VMEM((WLEN,), jnp.int32),
        pltpu.VMEM((GK, D), jnp.float32),
        pltpu.VMEM((GK, D), jnp.float32),
        pltpu.VMEM((GK,), jnp.int32),
        pltpu.VMEM((GK,), jnp.int32),
        pltpu.VMEM((ZR, D), jnp.float32),
        pltpu.SemaphoreType.DMA,
        pltpu.SemaphoreType.DMA,
    ]

    @functools.partial(pl.kernel, out_type=out_type, mesh=mesh,
                       scratch_types=scratch,
                       compiler_params=pltpu.CompilerParams(
                           needs_layout_passes=False))
    def k(tab_h, src_h, dst_h,
          out_h,
          acc_sh, src_v, dst_v, rows0_v, rows1_v, db0_v, db1_v, zb_v,
          sem0, sem1):
        cid = lax.axis_index("c")
        sid = lax.axis_index("s")
        wid = sid * NC + cid
        lo = wid * CPT
        hi = lo + CPT
        st = pl.multiple_of(jnp.minimum((lo // 8) * 8, NNZ - WLEN), 8)

        pltpu.sync_copy(src_h.at[pl.ds(st, WLEN)], src_v)
        pltpu.sync_copy(dst_h.at[pl.ds(st, WLEN)], dst_v)

        zf = jnp.zeros((L,), jnp.float32)

        def zero_zb(r, _):
            for k2 in range(D // L):
                zb_v[r, pl.ds(k2 * L, L)] = zf
            return _
        lax.fori_loop(0, ZR, zero_zb, None)

        def zero_acc(t, _):
            pltpu.sync_copy(zb_v, acc_sh.at[pl.ds(sid * RPS + t * ZR, ZR)])
            return _
        lax.fori_loop(0, ZFULL, zero_acc, None)
        if ZREM:
            pltpu.sync_copy(zb_v.at[pl.ds(0, ZREM)],
                            acc_sh.at[pl.ds(sid * RPS + ZFULL * ZR, ZREM)])
        plsc.subcore_barrier()

        iota = lax.iota(jnp.int32, L)

        def prep(g, db_b):
            gb = pl.multiple_of(g * GK, 8)
            for sub in range(GK // L):
                off = pl.multiple_of(gb + sub * L, 8)
                pos = st + off + iota
                m = (pos >= lo) & (pos < hi)
                dd = dst_v[pl.ds(off, L)]
                db_b[pl.ds(sub * L, L)] = jnp.where(m, dd, jnp.int32(A))

        def issue(g, rows_b, sem_b):
            gb = pl.multiple_of(g * GK, 8)
            return pltpu.async_copy(tab_h.at[src_v.at[pl.ds(gb, GK)]],
                                    rows_b, sem_b)

        def wait0():
            pltpu.make_async_copy(tab_h.at[pl.ds(0, GK)], rows0_v, sem0).wait()

        prep(0, db0_v)
        issue(0, rows0_v, sem0)

        def pbody(u, _):
            g1 = 2 * u + 1
            prep(g1, db1_v)
            cp1 = issue(g1, rows1_v, sem1)
            wait0()
            pltpu.sync_copy(rows0_v, acc_sh.at[db0_v], add=True)

            @pl.when(u < NP - 1)
            def _tail():
                prep(g1 + 1, db0_v)
                issue(g1 + 1, rows0_v, sem0)

            cp1.wait()
            pltpu.sync_copy(rows1_v, acc_sh.at[db1_v], add=True)
            return _
        lax.fori_loop(0, NP, pbody, None)

        plsc.subcore_barrier()
        pltpu.sync_copy(acc_sh.at[pl.ds(sid * RPS, RPS)],
                        out_h.at[cid, pl.ds(sid * RPS, RPS)])

    return k


def _r2_body(mp_ref, dei_ref, m_ref):
    nhe = m_ref.shape[0]
    m = mp_ref[0, :nhe, :] + mp_ref[1, :nhe, :]
    m_ref[...] = m * dei_ref[0, :][:, None]


def _r2_stage(m_p, dei, NHE, D):
    return pl.pallas_call(
        _r2_body,
        out_shape=jax.ShapeDtypeStruct((NHE, D), jnp.float32),
    )(m_p, dei)


def _f_body(xgp_ref, zgp_ref, dvi_ref, scale_ref, out_ref):
    n = out_ref.shape[0]
    xg = xgp_ref[0, :n, :] + xgp_ref[1, :n, :]
    z = zgp_ref[0, :n, :] + zgp_ref[1, :n, :]
    xhg = z * dvi_ref[0, :][:, None] * scale_ref[...]
    x1 = (xg + xhg) * 0.5
    out_ref[...] = jnp.where(x1 > 0, x1, jnp.exp(jnp.minimum(x1, 0.0)) - 1.0)


def _f_stage(xg_p, zg_p, dvi, scale_row, N, D):
    return pl.pallas_call(
        _f_body,
        out_shape=jax.ShapeDtypeStruct((N, D), jnp.float32),
    )(xg_p, zg_p, dvi, scale_row)


def kernel(X, edge_index, hg_v_idx, hg_e_idx, num_hyperedges, W_theta, b_theta,
           a_src, a_dst):
    N, D = X.shape
    E = edge_index.shape[1]
    NNZ = hg_v_idx.shape[0]
    NHE = 5000
    NPAD = _ceil_to(N + 1, NS * 8)
    HPAD = _ceil_to(NHE + 1, NS * 8)

    scale_row = jnp.broadcast_to(
        (jnp.asarray(num_hyperedges, jnp.float32) / NHE)[None, None], (1, D))

    Xo, s, c = _dense_stage(X, W_theta, b_theta, a_src, a_dst)
    e_src = edge_index[0]
    e_dst = edge_index[1]

    k2 = _build_edge_scalar(N, E, NNZ, NHE)
    ex, den_p, dv_p, de_p = k2(s[0, 0], s[1, 0], c[0, :L], e_src, e_dst,
                               hg_v_idx, hg_e_idx)

    rden, dvi, dei, Y = _r1_stage(den_p, dv_p, de_p, Xo)

    k3 = _build_edge_agg(N, E, D)
    xg_p = k3(Xo, ex, rden, e_src, e_dst)

    k4 = _build_gather_scatter(N, NHE, HPAD, NNZ, D)
    m_p = k4(Y, hg_v_idx, hg_e_idx)

    M = _r2_stage(m_p, dei, NHE, D)

    k5 = _build_gather_scatter(NHE, N, NPAD, NNZ, D)
    zg_p = k5(M, hg_e_idx, hg_v_idx)

    return _f_stage(xg_p, zg_p, dvi, scale_row, N, D)
